# Initial kernel scaffold; baseline (speedup 1.0000x reference)
import jax
import jax.numpy as jnp
from jax.experimental import pallas as pl


def kernel(x, adj_t, W1l, b1, W1r, W2l, b2, W2r, W3l, b3, W3r):
    raise NotImplementedError("write your pallas kernel here")


# speedup vs baseline: 7.2383x; 7.2383x over previous
import functools

import jax
import jax.numpy as jnp
from jax import lax
from jax.experimental import pallas as pl
from jax.experimental.pallas import tpu as pltpu
from jax.experimental.pallas import tpu_sc as plsc

_N = 10000
_D = 128
_E = 320000
_NC = 2
_NS = 16
_NW = _NC * _NS
_BATCH = 64
_NB = -(-(_E // _NW) // _BATCH)
_EPW = _NB * _BATCH
_PADE = _NW * _EPW - _E
_NP = 10240
_ZROWS = _NP // _NS
_DEGP = _N + 16


def _sc_agg(table, srcs, dsts, zeros, compute_deg):
  mesh = plsc.VectorSubcoreMesh(core_axis_name="c", subcore_axis_name="s")
  out_type = [jax.ShapeDtypeStruct((_NC, _NP, _D), jnp.float32)]
  if compute_deg:
    out_type.append(jax.ShapeDtypeStruct((_NW, 1, _DEGP), jnp.float32))
  scratch = [
      pltpu.VMEM_SHARED((_NP, _D), jnp.float32),
      pltpu.VMEM((_NB, _BATCH), jnp.int32),
      pltpu.VMEM((_NB, _BATCH), jnp.int32),
      pltpu.VMEM((2, _BATCH, _D), jnp.float32),
      pltpu.SemaphoreType.DMA,
      pltpu.SemaphoreType.DMA,
  ]
  if compute_deg:
    scratch.append(pltpu.VMEM((_DEGP,), jnp.float32))

  def body(table_ref, src_ref, dst_ref, zero_ref, *rest):
    if compute_deg:
      part_ref, degp_ref, acc, src_v, dst_v, rows, sem0, sem1, degloc = rest
    else:
      part_ref, acc, src_v, dst_v, rows, sem0, sem1 = rest
    core = lax.axis_index("c")
    sub = lax.axis_index("s")
    wid = core * _NS + sub
    pltpu.sync_copy(src_ref.at[wid], src_v)
    pltpu.sync_copy(dst_ref.at[wid], dst_v)
    pltpu.sync_copy(zero_ref.at[pl.ds(sub * _ZROWS, _ZROWS)],
                    acc.at[pl.ds(sub * _ZROWS, _ZROWS)])

    if compute_deg:
      zf = jnp.zeros((16,), jnp.float32)
      def zstep(i, c):
        degloc[pl.ds(i * 16, 16)] = zf
        return c
      lax.fori_loop(0, _DEGP // 16, zstep, 0)
      onef = jnp.ones((16,), jnp.float32)
      g = _BATCH // 16
      def dstep(i, c):
        d = dst_v[i // g, pl.ds((i % g) * 16, 16)]
        plsc.addupdate_scatter(degloc, [d], onef)
        return c
      lax.fori_loop(0, _NB * g, dstep, 0)
      pltpu.sync_copy(degloc, degp_ref.at[wid, 0])

    plsc.subcore_barrier()
    sems = (sem0, sem1)
    handles = [None, None]
    handles[0] = pltpu.async_copy(table_ref.at[src_v.at[0]], rows.at[0], sem0)
    for j in range(_NB):
      s = j % 2
      if j + 1 < _NB:
        ns = (j + 1) % 2
        handles[ns] = pltpu.async_copy(
            table_ref.at[src_v.at[j + 1]], rows.at[ns], sems[ns])
      handles[s].wait()
      pltpu.sync_copy(rows.at[s], acc.at[dst_v.at[j]], add=True)
    plsc.subcore_barrier()
    pltpu.sync_copy(acc.at[pl.ds(sub * _ZROWS, _ZROWS)],
                    part_ref.at[core, pl.ds(sub * _ZROWS, _ZROWS)])

  fn = pl.kernel(body, out_type=tuple(out_type), mesh=mesh,
                 scratch_types=tuple(scratch),
                 compiler_params=pltpu.CompilerParams(
                     needs_layout_passes=False,
                     use_tc_tiling_on_sc=False))
  return fn(table, srcs, dsts, zeros)


def _combine1(parts, degp_t, x, wl, wr, b):
  def body(p_ref, degp_ref, x_ref, wl_ref, wr_ref, b_ref, h_ref, degc_ref):
    deg = jnp.sum(degp_ref[...][:_N, :], axis=1, keepdims=True)
    degc = jnp.maximum(deg, 1.0)
    s = p_ref[0, :_N, :] + p_ref[1, :_N, :]
    h = jnp.dot(s / degc, wl_ref[...], preferred_element_type=jnp.float32)
    h = h + b_ref[...] + jnp.dot(x_ref[...], wr_ref[...],
                                 preferred_element_type=jnp.float32)
    h_ref[...] = jnp.maximum(h, 0.0)
    degc_ref[...] = degc
  return pl.pallas_call(
      body,
      out_shape=(jax.ShapeDtypeStruct((_N, _D), jnp.float32),
                 jax.ShapeDtypeStruct((_N, 1), jnp.float32)),
  )(parts, degp_t, x, wl, wr, b)


def _combine2(parts, degc, hprev, wl, wr, b):
  def body(p_ref, degc_ref, hp_ref, wl_ref, wr_ref, b_ref, h_ref):
    s = p_ref[0, :_N, :] + p_ref[1, :_N, :]
    h = jnp.dot(s / degc_ref[...], wl_ref[...],
                preferred_element_type=jnp.float32)
    h = h + b_ref[...] + jnp.dot(hp_ref[...], wr_ref[...],
                                 preferred_element_type=jnp.float32)
    h_ref[...] = jnp.maximum(h, 0.0)
  return pl.pallas_call(
      body,
      out_shape=jax.ShapeDtypeStruct((_N, _D), jnp.float32),
  )(parts, degc, hprev, wl, wr, b)


def _combine3(parts, degc, h2, wl, wr, b, h1):
  def body(p_ref, degc_ref, h2_ref, wl_ref, wr_ref, b_ref, h1_ref, o_ref):
    s = p_ref[0, :_N, :] + p_ref[1, :_N, :]
    h3 = jnp.dot(s / degc_ref[...], wl_ref[...],
                 preferred_element_type=jnp.float32)
    h3 = h3 + b_ref[...] + jnp.dot(h2_ref[...], wr_ref[...],
                                   preferred_element_type=jnp.float32)
    o_ref[...] = jnp.maximum(jnp.maximum(h1_ref[...], h2_ref[...]), h3)
  return pl.pallas_call(
      body,
      out_shape=jax.ShapeDtypeStruct((_N, _D), jnp.float32),
  )(parts, degc, h2, wl, wr, b, h1)


def kernel(x, adj_t, W1l, b1, W1r, W2l, b2, W2r, W3l, b3, W3r):
  src = adj_t[0]
  dst = adj_t[1]
  srcs = jnp.concatenate([src, jnp.zeros((_PADE,), jnp.int32)])
  srcs = srcs.reshape(_NW, _NB, _BATCH)
  dsts = jnp.concatenate([dst, jnp.full((_PADE,), _N, jnp.int32)])
  dsts = dsts.reshape(_NW, _NB, _BATCH)
  zeros = jnp.zeros((_NP, _D), jnp.float32)
  b1r, b2r, b3r = b1.reshape(1, _D), b2.reshape(1, _D), b3.reshape(1, _D)

  p1, degp = _sc_agg(x, srcs, dsts, zeros, True)
  degp_t = degp.reshape(_NW, _DEGP).T
  h1, degc = _combine1(p1, degp_t, x, W1l, W1r, b1r)
  (p2,) = _sc_agg(h1, srcs, dsts, zeros, False)
  h2 = _combine2(p2, degc, h1, W2l, W2r, b2r)
  (p3,) = _sc_agg(h2, srcs, dsts, zeros, False)
  return _combine3(p3, degc, h2, W3l, W3r, b3r, h1)
